# baseline (speedup 1.0000x reference)
import functools

import jax
import jax.numpy as jnp
from jax import lax
from jax.experimental import pallas as pl
from jax.experimental.pallas import tpu as pltpu
from jax.experimental.pallas import tpu_sc as plsc

B, S, E = 4096, 50, 128
H, O = 64, 10
NC, NS = 2, 16
NW = NC * NS
BPW = B // NW
PAIR = 2
CSZ = PAIR * S
CH = BPW // PAIR
EV = E // 16
NBUF = 8


def _rowsum(bufslot, base):
    UNR = 5

    def jb(j, carry):
        r = base + j * UNR
        vs = list(carry)
        for u in range(UNR):
            for e in range(EV):
                vs[e] = vs[e] + bufslot[r + u, pl.ds(e * 16, 16)]
        return tuple(vs)

    z = tuple(jnp.zeros((16,), jnp.float32) for _ in range(EV))
    return lax.fori_loop(0, S // UNR, jb, z)


def _sc_gather_sum(xg, table, dummy):
    mesh = plsc.VectorSubcoreMesh(core_axis_name="c", subcore_axis_name="s")

    @functools.partial(
        pl.kernel,
        out_type=jax.ShapeDtypeStruct((NW, BPW, E), jnp.float32),
        mesh=mesh,
        scratch_types=[
            pltpu.VMEM((CH, CSZ), jnp.int32),
            pltpu.VMEM((NBUF, CSZ, E), jnp.float32),
            pltpu.VMEM((BPW, E), jnp.float32),
            [pltpu.SemaphoreType.DMA] * NBUF,
        ],
    )
    def run(x_hbm, table_hbm, dummy_hbm, out_hbm, xv, buf, acc, sems):
        wid = lax.axis_index("s") * NC + lax.axis_index("c")
        pltpu.sync_copy(x_hbm.at[wid], xv)
        for slot in range(NBUF):
            pltpu.async_copy(table_hbm.at[xv.at[slot]], buf.at[slot], sems[slot])

        def step(slot, ci):
            pltpu.make_async_copy(
                dummy_hbm, buf.at[slot], sems[slot]
            ).wait()
            for p in range(PAIR):
                vs = _rowsum(buf.at[slot], p * S)
                row = ci * PAIR + p
                for e in range(EV):
                    acc[row, pl.ds(e * 16, 16)] = vs[e]

            @pl.when(ci + NBUF < CH)
            def _():
                pltpu.async_copy(
                    table_hbm.at[xv.at[ci + NBUF]], buf.at[slot], sems[slot]
                )

        def outer(i, carry):
            for slot in range(NBUF):
                step(slot, i * NBUF + slot)
            return carry

        lax.fori_loop(0, CH // NBUF, outer, 0)
        pltpu.sync_copy(acc, out_hbm.at[wid])

    return run(xg, table, dummy)


def _tc_mlp(sums, lens, w1t, b1, w2t, b2):
    def body(s_ref, l_ref, w1_ref, b1_ref, w2_ref, b2_ref, out_ref):
        rep = s_ref[:] / l_ref[:]
        h = jnp.dot(rep, w1_ref[:], preferred_element_type=jnp.float32)
        h = jnp.maximum(h + b1_ref[:], 0.0)
        out_ref[:] = (
            jnp.dot(h, w2_ref[:], preferred_element_type=jnp.float32) + b2_ref[:]
        )

    return pl.pallas_call(
        body,
        out_shape=jax.ShapeDtypeStruct((B, O), jnp.float32),
    )(sums, lens, w1t, b1, w2t, b2)


def kernel(x, lengths, table, W1, b1, W2, b2):
    xg = x.reshape(NW, CH, CSZ)
    dummy = jnp.zeros((CSZ, E), jnp.float32)
    sums = _sc_gather_sum(xg, table, dummy).reshape(B, E)
    lens = lengths.astype(jnp.float32).reshape(B, 1)
    return _tc_mlp(sums, lens, W1.T, b1.reshape(1, H), W2.T, b2.reshape(1, O))


# speedup vs baseline: 1.0238x; 1.0238x over previous
import functools

import jax
import jax.numpy as jnp
from jax import lax
from jax.experimental import pallas as pl
from jax.experimental.pallas import tpu as pltpu
from jax.experimental.pallas import tpu_sc as plsc

B, S, E = 4096, 50, 128
H, O = 64, 10
NC, NS = 2, 16
NW = NC * NS
BPW = B // NW
PAIR = 2
CSZ = PAIR * S
CH = BPW // PAIR
EV = E // 16
NBUF = 4


def _rowsum(bufslot, base):
    UNR = 5

    def jb(j, carry):
        r = base + j * UNR
        vs = list(carry)
        for u in range(UNR):
            for e in range(EV):
                vs[e] = vs[e] + bufslot[r + u, pl.ds(e * 16, 16)]
        return tuple(vs)

    z = tuple(jnp.zeros((16,), jnp.float32) for _ in range(EV))
    return lax.fori_loop(0, S // UNR, jb, z)


def _sc_gather_sum(xg, table, dummy):
    mesh = plsc.VectorSubcoreMesh(core_axis_name="c", subcore_axis_name="s")

    @functools.partial(
        pl.kernel,
        out_type=jax.ShapeDtypeStruct((NW, BPW, E), jnp.float32),
        mesh=mesh,
        scratch_types=[
            pltpu.VMEM((CH, CSZ), jnp.int32),
            pltpu.VMEM((NBUF, CSZ, E), jnp.float32),
            pltpu.VMEM((BPW, E), jnp.float32),
            [pltpu.SemaphoreType.DMA] * NBUF,
        ],
    )
    def run(x_hbm, table_hbm, dummy_hbm, out_hbm, xv, buf, acc, sems):
        wid = lax.axis_index("s") * NC + lax.axis_index("c")
        pltpu.sync_copy(x_hbm.at[wid], xv)
        for slot in range(NBUF):
            pltpu.async_copy(table_hbm.at[xv.at[slot]], buf.at[slot], sems[slot])

        def step(slot, ci):
            pltpu.make_async_copy(
                dummy_hbm, buf.at[slot], sems[slot]
            ).wait()
            for p in range(PAIR):
                vs = _rowsum(buf.at[slot], p * S)
                row = ci * PAIR + p
                for e in range(EV):
                    acc[row, pl.ds(e * 16, 16)] = vs[e]

            @pl.when(ci + NBUF < CH)
            def _():
                pltpu.async_copy(
                    table_hbm.at[xv.at[ci + NBUF]], buf.at[slot], sems[slot]
                )

        def outer(i, carry):
            for slot in range(NBUF):
                step(slot, i * NBUF + slot)
            return carry

        lax.fori_loop(0, CH // NBUF, outer, 0)
        pltpu.sync_copy(acc, out_hbm.at[wid])

    return run(xg, table, dummy)


def _tc_mlp(sums, lens, w1t, b1, w2t, b2):
    def body(s_ref, l_ref, w1_ref, b1_ref, w2_ref, b2_ref, out_ref):
        rep = s_ref[:] / l_ref[:]
        h = jnp.dot(rep, w1_ref[:], preferred_element_type=jnp.float32)
        h = jnp.maximum(h + b1_ref[:], 0.0)
        out_ref[:] = (
            jnp.dot(h, w2_ref[:], preferred_element_type=jnp.float32) + b2_ref[:]
        )

    return pl.pallas_call(
        body,
        out_shape=jax.ShapeDtypeStruct((B, O), jnp.float32),
    )(sums, lens, w1t, b1, w2t, b2)


def kernel(x, lengths, table, W1, b1, W2, b2):
    xg = x.reshape(NW, CH, CSZ)
    dummy = jnp.zeros((CSZ, E), jnp.float32)
    sums = _sc_gather_sum(xg, table, dummy).reshape(B, E)
    lens = lengths.astype(jnp.float32).reshape(B, 1)
    return _tc_mlp(sums, lens, W1.T, b1.reshape(1, H), W2.T, b2.reshape(1, O))


# speedup vs baseline: 1.0509x; 1.0264x over previous
import functools

import jax
import jax.numpy as jnp
from jax import lax
from jax.experimental import pallas as pl
from jax.experimental.pallas import tpu as pltpu
from jax.experimental.pallas import tpu_sc as plsc

B, S, E = 4096, 50, 128
H, O = 64, 10
NC, NS = 2, 16
NW = NC * NS
BPW = B // NW
PAIR = 1
CSZ = PAIR * S
CH = BPW // PAIR
EV = E // 16
NBUF = 8


def _rowsum(bufslot, base):
    UNR = 5

    def jb(j, carry):
        r = base + j * UNR
        vs = list(carry)
        for u in range(UNR):
            for e in range(EV):
                vs[e] = vs[e] + bufslot[r + u, pl.ds(e * 16, 16)]
        return tuple(vs)

    z = tuple(jnp.zeros((16,), jnp.float32) for _ in range(EV))
    return lax.fori_loop(0, S // UNR, jb, z)


def _sc_gather_sum(xg, table, dummy):
    mesh = plsc.VectorSubcoreMesh(core_axis_name="c", subcore_axis_name="s")

    @functools.partial(
        pl.kernel,
        out_type=jax.ShapeDtypeStruct((NW, BPW, E), jnp.float32),
        mesh=mesh,
        scratch_types=[
            pltpu.VMEM((CH, CSZ), jnp.int32),
            pltpu.VMEM((NBUF, CSZ, E), jnp.float32),
            pltpu.VMEM((BPW, E), jnp.float32),
            [pltpu.SemaphoreType.DMA] * NBUF,
        ],
    )
    def run(x_hbm, table_hbm, dummy_hbm, out_hbm, xv, buf, acc, sems):
        wid = lax.axis_index("s") * NC + lax.axis_index("c")
        pltpu.sync_copy(x_hbm.at[wid], xv)
        for slot in range(NBUF):
            pltpu.async_copy(table_hbm.at[xv.at[slot]], buf.at[slot], sems[slot])

        def step(slot, ci):
            pltpu.make_async_copy(
                dummy_hbm, buf.at[slot], sems[slot]
            ).wait()
            for p in range(PAIR):
                vs = _rowsum(buf.at[slot], p * S)
                row = ci * PAIR + p
                for e in range(EV):
                    acc[row, pl.ds(e * 16, 16)] = vs[e]

            @pl.when(ci + NBUF < CH)
            def _():
                pltpu.async_copy(
                    table_hbm.at[xv.at[ci + NBUF]], buf.at[slot], sems[slot]
                )

        def outer(i, carry):
            for slot in range(NBUF):
                step(slot, i * NBUF + slot)
            return carry

        lax.fori_loop(0, CH // NBUF, outer, 0)
        pltpu.sync_copy(acc, out_hbm.at[wid])

    return run(xg, table, dummy)


def _tc_mlp(sums, lens, w1t, b1, w2t, b2):
    def body(s_ref, l_ref, w1_ref, b1_ref, w2_ref, b2_ref, out_ref):
        rep = s_ref[:] / l_ref[:]
        h = jnp.dot(rep, w1_ref[:], preferred_element_type=jnp.float32)
        h = jnp.maximum(h + b1_ref[:], 0.0)
        out_ref[:] = (
            jnp.dot(h, w2_ref[:], preferred_element_type=jnp.float32) + b2_ref[:]
        )

    return pl.pallas_call(
        body,
        out_shape=jax.ShapeDtypeStruct((B, O), jnp.float32),
    )(sums, lens, w1t, b1, w2t, b2)


def kernel(x, lengths, table, W1, b1, W2, b2):
    xg = x.reshape(NW, CH, CSZ)
    dummy = jnp.zeros((CSZ, E), jnp.float32)
    sums = _sc_gather_sum(xg, table, dummy).reshape(B, E)
    lens = lengths.astype(jnp.float32).reshape(B, 1)
    return _tc_mlp(sums, lens, W1.T, b1.reshape(1, H), W2.T, b2.reshape(1, O))
